# baseline (speedup 1.0000x reference)
import functools

import jax
import jax.numpy as jnp
from jax.experimental import pallas as pl
from jax.experimental.pallas import tpu as pltpu

B, N, D, G, K = 4, 1024, 128, 4, 3
TILE_M = 256


def _layer_body(ef_ref, xx_ref, w_ref, b_ref, o_ref):
    e = ef_ref[0].astype(jnp.float32)
    agg = jnp.dot(e, xx_ref[0], preferred_element_type=jnp.float32)
    h = jnp.dot(agg, w_ref[0], preferred_element_type=jnp.float32) + b_ref[0]
    o_ref[0] = jnp.maximum(h, 0.0)


def _expand(v):
    z = jnp.zeros((B, N, 1, D), v.dtype)
    return jnp.concatenate([z, v[:, :, None, :], v[:, :, None, :]], axis=2
                           ).reshape(B, N * K, D)


@functools.partial(jax.jit, static_argnames=("interpret",))
def _layer(ef, xx, w, b, interpret=False):
    num_m = N // TILE_M
    return pl.pallas_call(
        _layer_body,
        grid=(B, num_m),
        in_specs=[
            pl.BlockSpec((1, TILE_M, N * K), lambda bb, m: (bb, m, 0)),
            pl.BlockSpec((1, N * K, D), lambda bb, m: (bb, 0, 0)),
            pl.BlockSpec((1, D, D), lambda bb, m: (bb % G, 0, 0)),
            pl.BlockSpec((1, 1, D), lambda bb, m: (bb % G, 0, 0)),
        ],
        out_specs=pl.BlockSpec((1, TILE_M, D), lambda bb, m: (bb, m, 0)),
        out_shape=jax.ShapeDtypeStruct((B, N, D), jnp.float32),
        compiler_params=pltpu.CompilerParams(
            dimension_semantics=("parallel", "parallel"),
        ),
        interpret=interpret,
    )(ef, xx, w, b)


def kernel(t, x, E, W1, b1, W2, b2, interpret=False):
    ef = E.reshape(B, N, N * K)
    h1 = _layer(ef, _expand(x), W1, b1.reshape(G, 1, D), interpret=interpret)
    h2 = _layer(ef, _expand(h1), W2, b2.reshape(G, 1, D), interpret=interpret)
    return h2


# speedup vs baseline: 1.2317x; 1.2317x over previous
import jax
import jax.numpy as jnp
from jax.experimental import pallas as pl
from jax.experimental.pallas import tpu as pltpu

B, N, D, G, K = 4, 1024, 128, 4, 3
TILE_M = 256
NM = N // TILE_M


def _rep3(v):
    return jnp.broadcast_to(v[:, None, :], (v.shape[0], K, v.shape[1])
                            ).reshape(v.shape[0] * K, v.shape[1])


def _body(e_ref, x_ref, w_ref, bias_ref, o_ref, ebf_ref, xx_ref, hh_ref):
    l = pl.program_id(1)
    m = pl.program_id(2)

    @pl.when(l == 0)
    def _layer0():
        @pl.when(m == 0)
        def _build_xx():
            xx_ref[...] = _rep3(x_ref[0].astype(jnp.bfloat16))

        colmask = (jax.lax.broadcasted_iota(jnp.int32, (1, N * K), 1) % K != 0)
        ebf = e_ref[0].astype(jnp.bfloat16) * colmask.astype(jnp.bfloat16)
        ebf_ref[pl.ds(m * TILE_M, TILE_M), :] = ebf
        agg = jnp.dot(ebf, xx_ref[...], preferred_element_type=jnp.float32)
        h = jnp.dot(agg, w_ref[0, 0], preferred_element_type=jnp.float32)
        h = jnp.maximum(h + bias_ref[0, 0], 0.0)
        hh_ref[pl.ds(m * TILE_M * K, TILE_M * K), :] = _rep3(h.astype(jnp.bfloat16))

    @pl.when(l == 1)
    def _layer1():
        ebf = ebf_ref[pl.ds(m * TILE_M, TILE_M), :]
        agg = jnp.dot(ebf, hh_ref[...], preferred_element_type=jnp.float32)
        h = jnp.dot(agg, w_ref[0, 0], preferred_element_type=jnp.float32)
        o_ref[0] = jnp.maximum(h + bias_ref[0, 0], 0.0)


def kernel(t, x, E, W1, b1, W2, b2, interpret=False):
    ef = E.reshape(B, N, N * K)
    wc = jnp.stack([W1, W2])
    bc = jnp.stack([b1, b2]).reshape(2, G, 1, D)
    return pl.pallas_call(
        _body,
        grid=(B, 2, NM),
        in_specs=[
            pl.BlockSpec((1, TILE_M, N * K),
                         lambda b, l, m: (b, m * (1 - l) + (NM - 1) * l, 0)),
            pl.BlockSpec((1, N, D), lambda b, l, m: (b, 0, 0)),
            pl.BlockSpec((1, 1, D, D), lambda b, l, m: (l, b % G, 0, 0)),
            pl.BlockSpec((1, 1, 1, D), lambda b, l, m: (l, b % G, 0, 0)),
        ],
        out_specs=pl.BlockSpec((1, TILE_M, D), lambda b, l, m: (b, m * l, 0)),
        out_shape=jax.ShapeDtypeStruct((B, N, D), jnp.float32),
        scratch_shapes=[
            pltpu.VMEM((N, N * K), jnp.bfloat16),
            pltpu.VMEM((N * K, D), jnp.bfloat16),
            pltpu.VMEM((N * K, D), jnp.bfloat16),
        ],
        compiler_params=pltpu.CompilerParams(
            dimension_semantics=("arbitrary", "arbitrary", "arbitrary"),
        ),
        interpret=interpret,
    )(ef, x, wc, bc)


# speedup vs baseline: 1.2776x; 1.0373x over previous
import numpy as np

import jax
import jax.numpy as jnp
from jax.experimental import pallas as pl
from jax.experimental.pallas import tpu as pltpu

B, N, D, G, K = 4, 1024, 128, 4, 3
TILE_M = 256
NM = N // TILE_M


def _body(e_ref, x_ref, r_ref, w_ref, bias_ref, o_ref, ebf_ref, xx_ref, hh_ref):
    l = pl.program_id(1)
    m = pl.program_id(2)

    @pl.when(l == 0)
    def _layer0():
        @pl.when(m == 0)
        def _build_xx():
            for i in range(NM):
                xt = x_ref[0, pl.ds(i * TILE_M, TILE_M), :].astype(jnp.bfloat16)
                xx_ref[pl.ds(i * TILE_M * K, TILE_M * K), :] = jnp.dot(
                    r_ref[...], xt, preferred_element_type=jnp.float32,
                ).astype(jnp.bfloat16)

        ebf = e_ref[0].astype(jnp.bfloat16)
        ebf_ref[pl.ds(m * TILE_M, TILE_M), :] = ebf
        agg = jnp.dot(ebf, xx_ref[...], preferred_element_type=jnp.float32)
        h = jnp.dot(agg, w_ref[0, 0], preferred_element_type=jnp.float32)
        h = jnp.maximum(h + bias_ref[0, 0], 0.0)
        hh_ref[pl.ds(m * TILE_M * K, TILE_M * K), :] = jnp.dot(
            r_ref[...], h.astype(jnp.bfloat16), preferred_element_type=jnp.float32,
        ).astype(jnp.bfloat16)

    @pl.when(l == 1)
    def _layer1():
        ebf = ebf_ref[pl.ds(m * TILE_M, TILE_M), :]
        agg = jnp.dot(ebf, hh_ref[...], preferred_element_type=jnp.float32)
        h = jnp.dot(agg, w_ref[0, 0], preferred_element_type=jnp.float32)
        o_ref[0] = jnp.maximum(h + bias_ref[0, 0], 0.0)


def kernel(t, x, E, W1, b1, W2, b2, interpret=False):
    ef = E.reshape(B, N, N * K)
    wc = jnp.stack([W1, W2])
    bc = jnp.stack([b1, b2]).reshape(2, G, 1, D)
    rnp = np.zeros((K * TILE_M, TILE_M), np.float32)
    rows = np.arange(K * TILE_M)
    rnp[rows, rows // K] = (rows % K != 0).astype(np.float32)
    rmat = jnp.asarray(rnp, dtype=jnp.bfloat16)
    return pl.pallas_call(
        _body,
        grid=(B, 2, NM),
        in_specs=[
            pl.BlockSpec((1, TILE_M, N * K),
                         lambda b, l, m: (b, m * (1 - l) + (NM - 1) * l, 0)),
            pl.BlockSpec((1, N, D), lambda b, l, m: (b, 0, 0)),
            pl.BlockSpec((K * TILE_M, TILE_M), lambda b, l, m: (0, 0)),
            pl.BlockSpec((1, 1, D, D), lambda b, l, m: (l, b % G, 0, 0)),
            pl.BlockSpec((1, 1, 1, D), lambda b, l, m: (l, b % G, 0, 0)),
        ],
        out_specs=pl.BlockSpec((1, TILE_M, D), lambda b, l, m: (b, m * l, 0)),
        out_shape=jax.ShapeDtypeStruct((B, N, D), jnp.float32),
        scratch_shapes=[
            pltpu.VMEM((N, N * K), jnp.bfloat16),
            pltpu.VMEM((N * K, D), jnp.bfloat16),
            pltpu.VMEM((N * K, D), jnp.bfloat16),
        ],
        compiler_params=pltpu.CompilerParams(
            dimension_semantics=("arbitrary", "arbitrary", "arbitrary"),
        ),
        interpret=interpret,
    )(ef, x, rmat, wc, bc)


# speedup vs baseline: 5.9426x; 4.6512x over previous
import jax
import jax.numpy as jnp
from jax.experimental import pallas as pl
from jax.experimental.pallas import tpu as pltpu

B, N, D, G, K = 4, 1024, 128, 4, 3
TILE_M = 256
NM = N // TILE_M


def _body(e1_ref, e2_ref, x_ref, w_ref, bias_ref, o_ref, abf_ref, h_ref):
    l = pl.program_id(1)
    m = pl.program_id(2)

    @pl.when(l == 0)
    def _layer0():
        a = (e1_ref[0, 0] + e2_ref[0, 0]).astype(jnp.bfloat16)
        abf_ref[pl.ds(m * TILE_M, TILE_M), :] = a
        agg = jnp.dot(a, x_ref[0].astype(jnp.bfloat16),
                      preferred_element_type=jnp.float32)
        h = jnp.dot(agg, w_ref[0, 0], preferred_element_type=jnp.float32)
        h = jnp.maximum(h + bias_ref[0, 0], 0.0)
        h_ref[pl.ds(m * TILE_M, TILE_M), :] = h.astype(jnp.bfloat16)

    @pl.when(l == 1)
    def _layer1():
        agg = jnp.dot(abf_ref[pl.ds(m * TILE_M, TILE_M), :], h_ref[...],
                      preferred_element_type=jnp.float32)
        h = jnp.dot(agg, w_ref[0, 0], preferred_element_type=jnp.float32)
        o_ref[0] = jnp.maximum(h + bias_ref[0, 0], 0.0)


def kernel(t, x, E, W1, b1, W2, b2, interpret=False):
    et = jnp.transpose(E, (0, 3, 1, 2))
    wc = jnp.stack([W1, W2])
    bc = jnp.stack([b1, b2]).reshape(2, G, 1, D)
    return pl.pallas_call(
        _body,
        grid=(B, 2, NM),
        in_specs=[
            pl.BlockSpec((1, 1, TILE_M, N),
                         lambda b, l, m: (b, 1, m * (1 - l) + (NM - 1) * l, 0)),
            pl.BlockSpec((1, 1, TILE_M, N),
                         lambda b, l, m: (b, 2, m * (1 - l) + (NM - 1) * l, 0)),
            pl.BlockSpec((1, N, D), lambda b, l, m: (b, 0, 0)),
            pl.BlockSpec((1, 1, D, D), lambda b, l, m: (l, b % G, 0, 0)),
            pl.BlockSpec((1, 1, 1, D), lambda b, l, m: (l, b % G, 0, 0)),
        ],
        out_specs=pl.BlockSpec((1, TILE_M, D), lambda b, l, m: (b, m * l, 0)),
        out_shape=jax.ShapeDtypeStruct((B, N, D), jnp.float32),
        scratch_shapes=[
            pltpu.VMEM((N, N), jnp.bfloat16),
            pltpu.VMEM((N, D), jnp.bfloat16),
        ],
        compiler_params=pltpu.CompilerParams(
            dimension_semantics=("arbitrary", "arbitrary", "arbitrary"),
        ),
        interpret=interpret,
    )(et, et, x, wc, bc)


# speedup vs baseline: 5.9514x; 1.0015x over previous
import jax
import jax.numpy as jnp
from jax.experimental import pallas as pl
from jax.experimental.pallas import tpu as pltpu

B, N, D, G, K = 4, 1024, 128, 4, 3
TILE_M = 256
NM = N // TILE_M


def _body(e1_ref, e2_ref, x_ref, w_ref, bias_ref, o_ref, abf_ref, h_ref):
    l = pl.program_id(1)
    m = pl.program_id(2)

    @pl.when(l == 0)
    def _layer0():
        a = (e1_ref[0, 0] + e2_ref[0, 0]).astype(jnp.bfloat16)
        abf_ref[pl.ds(m * TILE_M, TILE_M), :] = a
        agg = jnp.dot(a, x_ref[0].astype(jnp.bfloat16),
                      preferred_element_type=jnp.float32)
        h = jnp.dot(agg, w_ref[0, 0], preferred_element_type=jnp.float32)
        h = jnp.maximum(h + bias_ref[0, 0], 0.0)
        h_ref[pl.ds(m * TILE_M, TILE_M), :] = h.astype(jnp.bfloat16)

    @pl.when(l == 1)
    def _layer1():
        agg = jnp.dot(abf_ref[pl.ds(m * TILE_M, TILE_M), :], h_ref[...],
                      preferred_element_type=jnp.float32)
        h = jnp.dot(agg, w_ref[0, 0], preferred_element_type=jnp.float32)
        o_ref[0] = jnp.maximum(h + bias_ref[0, 0], 0.0)


def kernel(t, x, E, W1, b1, W2, b2, interpret=False):
    et = jnp.transpose(E, (0, 3, 1, 2))
    wc = jnp.stack([W1, W2])
    bc = jnp.stack([b1, b2]).reshape(2, G, 1, D)
    return pl.pallas_call(
        _body,
        grid=(B, 2, NM),
        in_specs=[
            pl.BlockSpec((1, 1, TILE_M, N),
                         lambda b, l, m: (b, 1, m * (1 - l) + (NM - 1) * l, 0)),
            pl.BlockSpec((1, 1, TILE_M, N),
                         lambda b, l, m: (b, 2, m * (1 - l) + (NM - 1) * l, 0)),
            pl.BlockSpec((1, N, D), lambda b, l, m: (b, 0, 0)),
            pl.BlockSpec((1, 1, D, D), lambda b, l, m: (l, b % G, 0, 0)),
            pl.BlockSpec((1, 1, 1, D), lambda b, l, m: (l, b % G, 0, 0)),
        ],
        out_specs=pl.BlockSpec((1, TILE_M, D), lambda b, l, m: (b, m * l, 0)),
        out_shape=jax.ShapeDtypeStruct((B, N, D), jnp.float32),
        scratch_shapes=[
            pltpu.VMEM((N, N), jnp.bfloat16),
            pltpu.VMEM((N, D), jnp.bfloat16),
        ],
        compiler_params=pltpu.CompilerParams(
            dimension_semantics=("parallel", "arbitrary", "arbitrary"),
        ),
        interpret=interpret,
    )(et, et, x, wc, bc)


# speedup vs baseline: 6.3360x; 1.0646x over previous
import jax
import jax.numpy as jnp
from jax.experimental import pallas as pl
from jax.experimental.pallas import tpu as pltpu

B, N, D, G, K = 4, 1024, 128, 4, 3
TILE_M = 256
NM = N // TILE_M


def _ecopy(e_hbm_ref, eraw_ref, sem_ref, b):
    return pltpu.make_async_copy(
        e_hbm_ref.at[b, pl.ds(N, 2 * N), :],
        eraw_ref.at[b % 2],
        sem_ref.at[b % 2],
    )


def _body(e_hbm_ref, x_ref, w_ref, bias_ref, o_ref, eraw_ref, abf_ref, h_ref,
          sem_ref):
    b = pl.program_id(0)
    l = pl.program_id(1)
    m = pl.program_id(2)

    @pl.when((l == 0) & (m == 0))
    def _stream_e():
        @pl.when(b == 0)
        def _first():
            _ecopy(e_hbm_ref, eraw_ref, sem_ref, 0).start()

        @pl.when(b < B - 1)
        def _prefetch_next():
            _ecopy(e_hbm_ref, eraw_ref, sem_ref, b + 1).start()

        _ecopy(e_hbm_ref, eraw_ref, sem_ref, b).wait()

    @pl.when(l == 0)
    def _layer0():
        slot = b % 2
        e1 = eraw_ref[slot, pl.ds(m * TILE_M, TILE_M), :]
        e2 = eraw_ref[slot, pl.ds(N + m * TILE_M, TILE_M), :]
        a = (e1 + e2).astype(jnp.bfloat16)
        abf_ref[pl.ds(m * TILE_M, TILE_M), :] = a
        agg = jnp.dot(a, x_ref[0].astype(jnp.bfloat16),
                      preferred_element_type=jnp.float32)
        h = jnp.dot(agg, w_ref[0, 0], preferred_element_type=jnp.float32)
        h = jnp.maximum(h + bias_ref[0, 0], 0.0)
        h_ref[pl.ds(m * TILE_M, TILE_M), :] = h.astype(jnp.bfloat16)

    @pl.when(l == 1)
    def _layer1():
        agg = jnp.dot(abf_ref[pl.ds(m * TILE_M, TILE_M), :], h_ref[...],
                      preferred_element_type=jnp.float32)
        h = jnp.dot(agg, w_ref[0, 0], preferred_element_type=jnp.float32)
        o_ref[0] = jnp.maximum(h + bias_ref[0, 0], 0.0)


def kernel(t, x, E, W1, b1, W2, b2, interpret=False):
    et = jnp.transpose(E, (0, 3, 1, 2)).reshape(B, K * N, N)
    wc = jnp.stack([W1, W2])
    bc = jnp.stack([b1, b2]).reshape(2, G, 1, D)
    return pl.pallas_call(
        _body,
        grid=(B, 2, NM),
        in_specs=[
            pl.BlockSpec(memory_space=pltpu.MemorySpace.HBM),
            pl.BlockSpec((1, N, D), lambda b, l, m: (b, 0, 0)),
            pl.BlockSpec((1, 1, D, D), lambda b, l, m: (l, b % G, 0, 0)),
            pl.BlockSpec((1, 1, 1, D), lambda b, l, m: (l, b % G, 0, 0)),
        ],
        out_specs=pl.BlockSpec((1, TILE_M, D), lambda b, l, m: (b, m * l, 0)),
        out_shape=jax.ShapeDtypeStruct((B, N, D), jnp.float32),
        scratch_shapes=[
            pltpu.VMEM((2, 2 * N, N), jnp.int32),
            pltpu.VMEM((N, N), jnp.bfloat16),
            pltpu.VMEM((N, D), jnp.bfloat16),
            pltpu.SemaphoreType.DMA((2,)),
        ],
        compiler_params=pltpu.CompilerParams(
            dimension_semantics=("arbitrary", "arbitrary", "arbitrary"),
        ),
        interpret=interpret,
    )(et, x, wc, bc)


# speedup vs baseline: 7.4740x; 1.1796x over previous
import jax
import jax.numpy as jnp
from jax.experimental import pallas as pl
from jax.experimental.pallas import tpu as pltpu

B, N, D, G, K = 4, 1024, 128, 4, 3
TILE_M = 512
NM = N // TILE_M


def _ecopy(e_hbm_ref, eraw_ref, sem_ref, b):
    return pltpu.make_async_copy(
        e_hbm_ref.at[b, pl.ds(N, 2 * N), :],
        eraw_ref.at[b % 2],
        sem_ref.at[b % 2],
    )


def _body(e_hbm_ref, x_ref, w_ref, bias_ref, o_ref, eraw_ref, abf_ref, h_ref,
          sem_ref):
    b = pl.program_id(0)
    l = pl.program_id(1)
    m = pl.program_id(2)

    @pl.when((l == 0) & (m == 0))
    def _stream_e():
        @pl.when(b == 0)
        def _first():
            _ecopy(e_hbm_ref, eraw_ref, sem_ref, 0).start()

        @pl.when(b < B - 1)
        def _prefetch_next():
            _ecopy(e_hbm_ref, eraw_ref, sem_ref, b + 1).start()

        _ecopy(e_hbm_ref, eraw_ref, sem_ref, b).wait()

    @pl.when(l == 0)
    def _layer0():
        slot = b % 2
        e1 = eraw_ref[slot, pl.ds(m * TILE_M, TILE_M), :]
        e2 = eraw_ref[slot, pl.ds(N + m * TILE_M, TILE_M), :]
        a = (e1 + e2).astype(jnp.bfloat16)
        abf_ref[pl.ds(m * TILE_M, TILE_M), :] = a
        agg = jnp.dot(a, x_ref[0].astype(jnp.bfloat16),
                      preferred_element_type=jnp.float32)
        h = jnp.dot(agg, w_ref[0, 0], preferred_element_type=jnp.float32)
        h = jnp.maximum(h + bias_ref[0, 0], 0.0)
        h_ref[pl.ds(m * TILE_M, TILE_M), :] = h.astype(jnp.bfloat16)

    @pl.when(l == 1)
    def _layer1():
        agg = jnp.dot(abf_ref[pl.ds(m * TILE_M, TILE_M), :], h_ref[...],
                      preferred_element_type=jnp.float32)
        h = jnp.dot(agg, w_ref[0, 0], preferred_element_type=jnp.float32)
        o_ref[0] = jnp.maximum(h + bias_ref[0, 0], 0.0)


def kernel(t, x, E, W1, b1, W2, b2, interpret=False):
    et = jnp.transpose(E, (0, 3, 1, 2)).reshape(B, K * N, N)
    wc = jnp.stack([W1, W2])
    bc = jnp.stack([b1, b2]).reshape(2, G, 1, D)
    return pl.pallas_call(
        _body,
        grid=(B, 2, NM),
        in_specs=[
            pl.BlockSpec(memory_space=pltpu.MemorySpace.HBM),
            pl.BlockSpec((1, N, D), lambda b, l, m: (b, 0, 0)),
            pl.BlockSpec((1, 1, D, D), lambda b, l, m: (l, b % G, 0, 0)),
            pl.BlockSpec((1, 1, 1, D), lambda b, l, m: (l, b % G, 0, 0)),
        ],
        out_specs=pl.BlockSpec((1, TILE_M, D), lambda b, l, m: (b, m * l, 0)),
        out_shape=jax.ShapeDtypeStruct((B, N, D), jnp.float32),
        scratch_shapes=[
            pltpu.VMEM((2, 2 * N, N), jnp.int32),
            pltpu.VMEM((N, N), jnp.bfloat16),
            pltpu.VMEM((N, D), jnp.bfloat16),
            pltpu.SemaphoreType.DMA((2,)),
        ],
        compiler_params=pltpu.CompilerParams(
            dimension_semantics=("arbitrary", "arbitrary", "arbitrary"),
        ),
        interpret=interpret,
    )(et, x, wc, bc)


# speedup vs baseline: 9.3813x; 1.2552x over previous
import jax
import jax.numpy as jnp
from jax.experimental import pallas as pl
from jax.experimental.pallas import tpu as pltpu

B, N, D, G, K = 4, 1024, 128, 4, 3
TILE_M = 1024
NM = N // TILE_M


def _ecopy(e_hbm_ref, eraw_ref, sem_ref, b):
    return pltpu.make_async_copy(
        e_hbm_ref.at[b, pl.ds(N, 2 * N), :],
        eraw_ref.at[b % 2],
        sem_ref.at[b % 2],
    )


def _body(e_hbm_ref, x_ref, w_ref, bias_ref, o_ref, eraw_ref, abf_ref, h_ref,
          sem_ref):
    b = pl.program_id(0)
    l = pl.program_id(1)
    m = pl.program_id(2)

    @pl.when((l == 0) & (m == 0))
    def _stream_e():
        @pl.when(b == 0)
        def _first():
            _ecopy(e_hbm_ref, eraw_ref, sem_ref, 0).start()

        @pl.when(b < B - 1)
        def _prefetch_next():
            _ecopy(e_hbm_ref, eraw_ref, sem_ref, b + 1).start()

        _ecopy(e_hbm_ref, eraw_ref, sem_ref, b).wait()

    @pl.when(l == 0)
    def _layer0():
        slot = b % 2
        e1 = eraw_ref[slot, pl.ds(m * TILE_M, TILE_M), :]
        e2 = eraw_ref[slot, pl.ds(N + m * TILE_M, TILE_M), :]
        a = (e1 + e2).astype(jnp.bfloat16)
        abf_ref[pl.ds(m * TILE_M, TILE_M), :] = a
        agg = jnp.dot(a, x_ref[0].astype(jnp.bfloat16),
                      preferred_element_type=jnp.float32)
        h = jnp.dot(agg, w_ref[0, 0], preferred_element_type=jnp.float32)
        h = jnp.maximum(h + bias_ref[0, 0], 0.0)
        h_ref[pl.ds(m * TILE_M, TILE_M), :] = h.astype(jnp.bfloat16)

    @pl.when(l == 1)
    def _layer1():
        agg = jnp.dot(abf_ref[pl.ds(m * TILE_M, TILE_M), :], h_ref[...],
                      preferred_element_type=jnp.float32)
        h = jnp.dot(agg, w_ref[0, 0], preferred_element_type=jnp.float32)
        o_ref[0] = jnp.maximum(h + bias_ref[0, 0], 0.0)


def kernel(t, x, E, W1, b1, W2, b2, interpret=False):
    et = jnp.transpose(E, (0, 3, 1, 2)).reshape(B, K * N, N)
    wc = jnp.stack([W1, W2])
    bc = jnp.stack([b1, b2]).reshape(2, G, 1, D)
    return pl.pallas_call(
        _body,
        grid=(B, 2, NM),
        in_specs=[
            pl.BlockSpec(memory_space=pltpu.MemorySpace.HBM),
            pl.BlockSpec((1, N, D), lambda b, l, m: (b, 0, 0)),
            pl.BlockSpec((1, 1, D, D), lambda b, l, m: (l, b % G, 0, 0)),
            pl.BlockSpec((1, 1, 1, D), lambda b, l, m: (l, b % G, 0, 0)),
        ],
        out_specs=pl.BlockSpec((1, TILE_M, D), lambda b, l, m: (b, m * l, 0)),
        out_shape=jax.ShapeDtypeStruct((B, N, D), jnp.float32),
        scratch_shapes=[
            pltpu.VMEM((2, 2 * N, N), jnp.int32),
            pltpu.VMEM((N, N), jnp.bfloat16),
            pltpu.VMEM((N, D), jnp.bfloat16),
            pltpu.SemaphoreType.DMA((2,)),
        ],
        compiler_params=pltpu.CompilerParams(
            dimension_semantics=("arbitrary", "arbitrary", "arbitrary"),
        ),
        interpret=interpret,
    )(et, x, wc, bc)


# speedup vs baseline: 11.0901x; 1.1821x over previous
import jax
import jax.numpy as jnp
from jax.experimental import pallas as pl
from jax.experimental.pallas import tpu as pltpu

B, N, D, G, K = 4, 1024, 128, 4, 3


def _ecopy(e_hbm_ref, eraw_ref, sem_ref, b):
    return pltpu.make_async_copy(
        e_hbm_ref.at[b, pl.ds(N, 2 * N), :],
        eraw_ref.at[b % 2],
        sem_ref.at[b % 2],
    )


def _body(e_hbm_ref, x_ref, w1_ref, b1_ref, w2_ref, b2_ref, o_ref,
          eraw_ref, sem_ref):
    b = pl.program_id(0)

    @pl.when(b == 0)
    def _first():
        _ecopy(e_hbm_ref, eraw_ref, sem_ref, 0).start()

    @pl.when(b < B - 1)
    def _prefetch_next():
        _ecopy(e_hbm_ref, eraw_ref, sem_ref, b + 1).start()

    _ecopy(e_hbm_ref, eraw_ref, sem_ref, b).wait()

    slot = b % 2
    a = (eraw_ref[slot, :N, :] + eraw_ref[slot, N:, :]).astype(jnp.bfloat16)
    agg = jnp.dot(a, x_ref[0].astype(jnp.bfloat16),
                  preferred_element_type=jnp.float32)
    h = jnp.dot(agg, w1_ref[0], preferred_element_type=jnp.float32)
    h = jnp.maximum(h + b1_ref[0], 0.0)
    agg = jnp.dot(a, h.astype(jnp.bfloat16), preferred_element_type=jnp.float32)
    h = jnp.dot(agg, w2_ref[0], preferred_element_type=jnp.float32)
    o_ref[0] = jnp.maximum(h + b2_ref[0], 0.0)


def kernel(t, x, E, W1, b1, W2, b2, interpret=False):
    et = jnp.transpose(E, (0, 3, 1, 2)).reshape(B, K * N, N)
    b1r = b1.reshape(G, 1, D)
    b2r = b2.reshape(G, 1, D)
    return pl.pallas_call(
        _body,
        grid=(B,),
        in_specs=[
            pl.BlockSpec(memory_space=pltpu.MemorySpace.HBM),
            pl.BlockSpec((1, N, D), lambda b: (b, 0, 0)),
            pl.BlockSpec((1, D, D), lambda b: (b % G, 0, 0)),
            pl.BlockSpec((1, 1, D), lambda b: (b % G, 0, 0)),
            pl.BlockSpec((1, D, D), lambda b: (b % G, 0, 0)),
            pl.BlockSpec((1, 1, D), lambda b: (b % G, 0, 0)),
        ],
        out_specs=pl.BlockSpec((1, N, D), lambda b: (b, 0, 0)),
        out_shape=jax.ShapeDtypeStruct((B, N, D), jnp.float32),
        scratch_shapes=[
            pltpu.VMEM((2, 2 * N, N), jnp.int32),
            pltpu.SemaphoreType.DMA((2,)),
        ],
        compiler_params=pltpu.CompilerParams(
            dimension_semantics=("arbitrary",),
        ),
        interpret=interpret,
    )(et, x, W1, b1r, W2, b2r)
